# baseline (speedup 1.0000x reference)
import jax
import jax.numpy as jnp
from jax import lax
from jax.experimental import pallas as pl
from jax.experimental.pallas import tpu as pltpu
from jax.experimental.pallas import tpu_sc as plsc

K = 19
NC, NS, L = 2, 16, 16
NW = NC * NS
CPT = 8
KC = K * CPT
CS = 4096
BSC = 3
BN_TC = 8192


def _sc_partial(feats3, labels2, b_lo, b_hi):
    B, C, N = feats3.shape
    cpb = N // CS
    nch = (b_hi - b_lo) * cpb

    def sc_body(feats_hbm, labels_hbm, out_hbm, cnt_hbm,
                lab_v, feats_v, acc_v, cacc_v, sums_v, csums_v, blk_v,
                sem_f0, sem_f1, sem_l0, sem_l1):
        sem_f = (sem_f0, sem_f1)
        sem_l = (sem_l0, sem_l1)
        wid = lax.axis_index("s") * NC + lax.axis_index("c")
        c0 = wid * CPT
        iota = lax.iota(jnp.int32, L)
        ones = jnp.ones((L,), jnp.float32)
        zeros = jnp.zeros((L,), jnp.float32)

        def chunk_src(c):
            b = b_lo + lax.div(c, cpb)
            off = lax.rem(c, cpb) * CS
            return (feats_hbm.at[b, pl.ds(c0, CPT), pl.ds(off, CS)],
                    labels_hbm.at[b, pl.ds(off, CS)])

        def _z(i, c):
            acc_v[pl.ds(i * L, L)] = zeros
            return c
        lax.fori_loop(0, KC, _z, 0)

        def _zc(i, c):
            cacc_v[pl.ds(i * L, L)] = zeros
            return c
        lax.fori_loop(0, K, _zc, 0)

        for p in range(2):
            fsrc, lsrc = chunk_src(jnp.int32(p))
            pltpu.async_copy(fsrc, feats_v.at[p], sem_f[p])
            pltpu.async_copy(lsrc, lab_v.at[p], sem_l[p])

        def _outer(t, cr):
            for p in range(2):
                c = t * 2 + p
                fsrc, lsrc = chunk_src(c)
                pltpu.make_async_copy(fsrc, feats_v.at[p], sem_f[p]).wait()
                pltpu.make_async_copy(lsrc, lab_v.at[p], sem_l[p]).wait()

                @plsc.parallel_loop(0, CS // L, unroll=2)
                def _grp(g):
                    pp = g * L
                    labs = lab_v[p, pl.ds(pp, L)]
                    base = labs * (CPT * L) + iota
                    plsc.addupdate_scatter(cacc_v, [labs * L + iota], ones)
                    for cl in range(CPT):
                        f = feats_v[p, cl, pl.ds(pp, L)]
                        plsc.addupdate_scatter(acc_v, [base + cl * L], f)

                @pl.when(c + 2 < nch)
                def _prefetch():
                    fsrc2, lsrc2 = chunk_src(c + 2)
                    pltpu.async_copy(fsrc2, feats_v.at[p], sem_f[p])
                    pltpu.async_copy(lsrc2, lab_v.at[p], sem_l[p])
            return cr
        lax.fori_loop(0, nch // 2, _outer, 0)

        lane_last = iota == (L - 1)

        def _fold(i, c):
            tot = plsc.cumsum(acc_v[pl.ds(i * L, L)])
            plsc.store_scatter(sums_v, [iota * 0 + i], tot, mask=lane_last)
            return c
        lax.fori_loop(0, KC, _fold, 0)

        def _foldc(i, c):
            tot = plsc.cumsum(cacc_v[pl.ds(i * L, L)])
            plsc.store_scatter(csums_v, [iota * 0 + i], tot, mask=lane_last)
            return c
        lax.fori_loop(0, K, _foldc, 0)

        for s in range(KC // L + 1):
            i = s * L + iota
            k_idx = lax.shift_right_logical(i, 3)
            cl_idx = jnp.bitwise_and(i, CPT - 1)
            valid = k_idx < K
            a = sums_v[pl.ds(s * L, L)]
            plsc.store_scatter(blk_v, [cl_idx, k_idx], a, mask=valid)

        pltpu.sync_copy(blk_v, out_hbm.at[pl.ds(c0, CPT), :])

        @pl.when(wid == 0)
        def _wcnt():
            pltpu.sync_copy(csums_v, cnt_hbm)

    mesh = plsc.VectorSubcoreMesh(core_axis_name="c", subcore_axis_name="s",
                                  num_cores=NC, num_subcores=NS)
    return pl.kernel(
        sc_body,
        out_type=[
            jax.ShapeDtypeStruct((C, K), jnp.float32),
            jax.ShapeDtypeStruct((2 * L,), jnp.float32),
        ],
        mesh=mesh,
        compiler_params=pltpu.CompilerParams(needs_layout_passes=False),
        scratch_types=[
            pltpu.VMEM((2, CS), jnp.int32),
            pltpu.VMEM((2, CPT, CS), jnp.float32),
            pltpu.VMEM((KC * L,), jnp.float32),
            pltpu.VMEM((K * L,), jnp.float32),
            pltpu.VMEM((KC + L,), jnp.float32),
            pltpu.VMEM((2 * L,), jnp.float32),
            pltpu.VMEM((CPT, K), jnp.float32),
            pltpu.SemaphoreType.DMA,
            pltpu.SemaphoreType.DMA,
            pltpu.SemaphoreType.DMA,
            pltpu.SemaphoreType.DMA,
        ],
    )(feats3, labels2)


def _tc_kernel_body(feats_ref, labels_ref, out_ref, cnt_ref, *, bn):
    b = pl.program_id(0)
    j = pl.program_id(1)

    feats = feats_ref[0]
    labels = labels_ref[0]
    classes = lax.broadcasted_iota(jnp.int32, (K, bn), 0)
    onehot = (labels == classes).astype(jnp.float32)

    partial = lax.dot_general(
        feats, onehot,
        dimension_numbers=(((1,), (1,)), ((), ())),
        preferred_element_type=jnp.float32,
    )
    cnt_partial = jnp.sum(onehot, axis=1).reshape(1, K)

    @pl.when((b == 0) & (j == 0))
    def _init():
        out_ref[...] = partial
        cnt_ref[...] = cnt_partial

    @pl.when((b > 0) | (j > 0))
    def _acc():
        out_ref[...] += partial
        cnt_ref[...] += cnt_partial


def _tc_partial(feats3, labels2, bt):
    B, C, N = feats3.shape
    nb = N // BN_TC
    labels3 = labels2.reshape(B * nb, 1, BN_TC)
    return pl.pallas_call(
        lambda f, l, o, c: _tc_kernel_body(f, l, o, c, bn=BN_TC),
        grid=(bt, nb),
        in_specs=[
            pl.BlockSpec((1, C, BN_TC), lambda b, j: (b, 0, j)),
            pl.BlockSpec((1, 1, BN_TC), lambda b, j: (b * nb + j, 0, 0)),
        ],
        out_specs=[
            pl.BlockSpec((C, K), lambda b, j: (0, 0)),
            pl.BlockSpec((1, K), lambda b, j: (0, 0)),
        ],
        out_shape=[
            jax.ShapeDtypeStruct((C, K), jnp.float32),
            jax.ShapeDtypeStruct((1, K), jnp.float32),
        ],
    )(feats3, labels3)


def _finalize_body(ts_ref, tcnt_ref, ss_ref, scnt_ref, proto_ref, cnt_ref):
    sums = ts_ref[...] + ss_ref[...]
    cnt = tcnt_ref[...] + scnt_ref[0, :K].reshape(1, K)
    denom = jnp.maximum(cnt, 1.0)
    proto_ref[...] = jnp.where(cnt > 0.0, sums / denom, jnp.zeros_like(sums))
    cnt_ref[...] = cnt


def kernel(features, labels):
    B, C, H, W = features.shape
    N = H * W
    feats3 = features.reshape(B, C, N)
    labels2 = labels.reshape(B, N)

    tc_sums, tc_cnts = _tc_partial(feats3, labels2, B - BSC)
    sc_sums, sc_cnts = _sc_partial(feats3, labels2, B - BSC, B)

    protos_t, counts = pl.pallas_call(
        _finalize_body,
        out_shape=[
            jax.ShapeDtypeStruct((C, K), jnp.float32),
            jax.ShapeDtypeStruct((1, K), jnp.float32),
        ],
    )(tc_sums, tc_cnts, sc_sums, sc_cnts.reshape(1, 2 * L))

    return protos_t.T, counts.reshape(K)


# speedup vs baseline: 1.0308x; 1.0308x over previous
import jax
import jax.numpy as jnp
from jax import lax
from jax.experimental import pallas as pl
from jax.experimental.pallas import tpu as pltpu
from jax.experimental.pallas import tpu_sc as plsc

K = 19
NC, NS, L = 2, 16, 16
NW = NC * NS
CPT = 8
KC = K * CPT
CS = 4096
BSC = 2
BN_TC = 8192


def _sc_partial(feats3, labels2, b_lo, b_hi):
    B, C, N = feats3.shape
    cpb = N // CS
    nch = (b_hi - b_lo) * cpb

    def sc_body(feats_hbm, labels_hbm, out_hbm, cnt_hbm,
                lab_v, feats_v, acc_v, cacc_v, sums_v, csums_v, blk_v,
                sem_f0, sem_f1, sem_l0, sem_l1):
        sem_f = (sem_f0, sem_f1)
        sem_l = (sem_l0, sem_l1)
        wid = lax.axis_index("s") * NC + lax.axis_index("c")
        c0 = wid * CPT
        iota = lax.iota(jnp.int32, L)
        ones = jnp.ones((L,), jnp.float32)
        zeros = jnp.zeros((L,), jnp.float32)

        def chunk_src(c):
            b = b_lo + lax.div(c, cpb)
            off = lax.rem(c, cpb) * CS
            return (feats_hbm.at[b, pl.ds(c0, CPT), pl.ds(off, CS)],
                    labels_hbm.at[b, pl.ds(off, CS)])

        def _z(i, c):
            acc_v[pl.ds(i * L, L)] = zeros
            return c
        lax.fori_loop(0, KC, _z, 0)

        def _zc(i, c):
            cacc_v[pl.ds(i * L, L)] = zeros
            return c
        lax.fori_loop(0, K, _zc, 0)

        for p in range(2):
            fsrc, lsrc = chunk_src(jnp.int32(p))
            pltpu.async_copy(fsrc, feats_v.at[p], sem_f[p])
            pltpu.async_copy(lsrc, lab_v.at[p], sem_l[p])

        def _outer(t, cr):
            for p in range(2):
                c = t * 2 + p
                fsrc, lsrc = chunk_src(c)
                pltpu.make_async_copy(fsrc, feats_v.at[p], sem_f[p]).wait()
                pltpu.make_async_copy(lsrc, lab_v.at[p], sem_l[p]).wait()

                @plsc.parallel_loop(0, CS // L, unroll=2)
                def _grp(g):
                    pp = g * L
                    labs = lab_v[p, pl.ds(pp, L)]
                    base = labs * (CPT * L) + iota
                    plsc.addupdate_scatter(cacc_v, [labs * L + iota], ones)
                    for cl in range(CPT):
                        f = feats_v[p, cl, pl.ds(pp, L)]
                        plsc.addupdate_scatter(acc_v, [base + cl * L], f)

                @pl.when(c + 2 < nch)
                def _prefetch():
                    fsrc2, lsrc2 = chunk_src(c + 2)
                    pltpu.async_copy(fsrc2, feats_v.at[p], sem_f[p])
                    pltpu.async_copy(lsrc2, lab_v.at[p], sem_l[p])
            return cr
        lax.fori_loop(0, nch // 2, _outer, 0)

        lane_last = iota == (L - 1)

        def _fold(i, c):
            tot = plsc.cumsum(acc_v[pl.ds(i * L, L)])
            plsc.store_scatter(sums_v, [iota * 0 + i], tot, mask=lane_last)
            return c
        lax.fori_loop(0, KC, _fold, 0)

        def _foldc(i, c):
            tot = plsc.cumsum(cacc_v[pl.ds(i * L, L)])
            plsc.store_scatter(csums_v, [iota * 0 + i], tot, mask=lane_last)
            return c
        lax.fori_loop(0, K, _foldc, 0)

        for s in range(KC // L + 1):
            i = s * L + iota
            k_idx = lax.shift_right_logical(i, 3)
            cl_idx = jnp.bitwise_and(i, CPT - 1)
            valid = k_idx < K
            a = sums_v[pl.ds(s * L, L)]
            plsc.store_scatter(blk_v, [cl_idx, k_idx], a, mask=valid)

        pltpu.sync_copy(blk_v, out_hbm.at[pl.ds(c0, CPT), :])

        @pl.when(wid == 0)
        def _wcnt():
            pltpu.sync_copy(csums_v, cnt_hbm)

    mesh = plsc.VectorSubcoreMesh(core_axis_name="c", subcore_axis_name="s",
                                  num_cores=NC, num_subcores=NS)
    return pl.kernel(
        sc_body,
        out_type=[
            jax.ShapeDtypeStruct((C, K), jnp.float32),
            jax.ShapeDtypeStruct((2 * L,), jnp.float32),
        ],
        mesh=mesh,
        compiler_params=pltpu.CompilerParams(needs_layout_passes=False),
        scratch_types=[
            pltpu.VMEM((2, CS), jnp.int32),
            pltpu.VMEM((2, CPT, CS), jnp.float32),
            pltpu.VMEM((KC * L,), jnp.float32),
            pltpu.VMEM((K * L,), jnp.float32),
            pltpu.VMEM((KC + L,), jnp.float32),
            pltpu.VMEM((2 * L,), jnp.float32),
            pltpu.VMEM((CPT, K), jnp.float32),
            pltpu.SemaphoreType.DMA,
            pltpu.SemaphoreType.DMA,
            pltpu.SemaphoreType.DMA,
            pltpu.SemaphoreType.DMA,
        ],
    )(feats3, labels2)


def _tc_kernel_body(feats_ref, labels_ref, out_ref, cnt_ref, *, bn):
    b = pl.program_id(0)
    j = pl.program_id(1)

    feats = feats_ref[0]
    labels = labels_ref[0]
    classes = lax.broadcasted_iota(jnp.int32, (K, bn), 0)
    onehot = (labels == classes).astype(jnp.float32)

    partial = lax.dot_general(
        feats, onehot,
        dimension_numbers=(((1,), (1,)), ((), ())),
        preferred_element_type=jnp.float32,
    )
    cnt_partial = jnp.sum(onehot, axis=1).reshape(1, K)

    @pl.when((b == 0) & (j == 0))
    def _init():
        out_ref[...] = partial
        cnt_ref[...] = cnt_partial

    @pl.when((b > 0) | (j > 0))
    def _acc():
        out_ref[...] += partial
        cnt_ref[...] += cnt_partial


def _tc_partial(feats3, labels2, bt):
    B, C, N = feats3.shape
    nb = N // BN_TC
    labels3 = labels2.reshape(B * nb, 1, BN_TC)
    return pl.pallas_call(
        lambda f, l, o, c: _tc_kernel_body(f, l, o, c, bn=BN_TC),
        grid=(bt, nb),
        in_specs=[
            pl.BlockSpec((1, C, BN_TC), lambda b, j: (b, 0, j)),
            pl.BlockSpec((1, 1, BN_TC), lambda b, j: (b * nb + j, 0, 0)),
        ],
        out_specs=[
            pl.BlockSpec((C, K), lambda b, j: (0, 0)),
            pl.BlockSpec((1, K), lambda b, j: (0, 0)),
        ],
        out_shape=[
            jax.ShapeDtypeStruct((C, K), jnp.float32),
            jax.ShapeDtypeStruct((1, K), jnp.float32),
        ],
    )(feats3, labels3)


def _finalize_body(ts_ref, tcnt_ref, ss_ref, scnt_ref, proto_ref, cnt_ref):
    sums = ts_ref[...] + ss_ref[...]
    cnt = tcnt_ref[...] + scnt_ref[0, :K].reshape(1, K)
    denom = jnp.maximum(cnt, 1.0)
    proto_ref[...] = jnp.where(cnt > 0.0, sums / denom, jnp.zeros_like(sums))
    cnt_ref[...] = cnt


def kernel(features, labels):
    B, C, H, W = features.shape
    N = H * W
    feats3 = features.reshape(B, C, N)
    labels2 = labels.reshape(B, N)

    tc_sums, tc_cnts = _tc_partial(feats3, labels2, B - BSC)
    sc_sums, sc_cnts = _sc_partial(feats3, labels2, B - BSC, B)

    protos_t, counts = pl.pallas_call(
        _finalize_body,
        out_shape=[
            jax.ShapeDtypeStruct((C, K), jnp.float32),
            jax.ShapeDtypeStruct((1, K), jnp.float32),
        ],
    )(tc_sums, tc_cnts, sc_sums, sc_cnts.reshape(1, 2 * L))

    return protos_t.T, counts.reshape(K)
